# baseline (speedup 1.0000x reference)
import functools
import jax
import jax.numpy as jnp
from jax import lax
from jax.experimental import pallas as pl
from jax.experimental.pallas import tpu as pltpu
from jax.experimental.pallas import tpu_sc as plsc

_NC = 2
_NS = 16
_NW = _NC * _NS
_ROWS_OUT = 64
_ROWS_PER_W = _ROWS_OUT // _NW
_D = 32768


@functools.partial(
    pl.kernel,
    mesh=plsc.VectorSubcoreMesh(core_axis_name="c", subcore_axis_name="s"),
    out_type=jax.ShapeDtypeStruct((_ROWS_OUT, _D), jnp.float32),
    scratch_types=[
        pltpu.SemaphoreType.DMA,
        pltpu.SemaphoreType.DMA,
    ],
)
def _sc_gather(u_hbm, out_hbm, sem0, sem1):
    wid = lax.axis_index("s") * _NC + lax.axis_index("c")
    base = wid * _ROWS_PER_W
    cp0 = pltpu.make_async_copy(
        u_hbm.at[4 * base], out_hbm.at[base], sem0)
    cp1 = pltpu.make_async_copy(
        u_hbm.at[4 * (base + 1)], out_hbm.at[base + 1], sem1)
    cp0.start()
    cp1.start()
    cp0.wait()
    cp1.wait()


def kernel(u):
    return _sc_gather(u)


# speedup vs baseline: 10.8500x; 10.8500x over previous
import functools
import jax
import jax.numpy as jnp
from jax import lax
from jax.experimental import pallas as pl
from jax.experimental.pallas import tpu as pltpu
from jax.experimental.pallas import tpu_sc as plsc

_NC = 2
_NS = 16
_NW = _NC * _NS
_ROWS_OUT = 64
_ROWS_PER_W = _ROWS_OUT // _NW
_D = 32768


@functools.partial(
    pl.kernel,
    mesh=plsc.VectorSubcoreMesh(core_axis_name="c", subcore_axis_name="s"),
    out_type=jax.ShapeDtypeStruct((_ROWS_OUT, _D), jnp.float32),
    scratch_types=[
        pltpu.VMEM((2, _D), jnp.float32),
        pltpu.SemaphoreType.DMA,
        pltpu.SemaphoreType.DMA,
        pltpu.SemaphoreType.DMA,
        pltpu.SemaphoreType.DMA,
    ],
)
def _sc_gather(u_hbm, out_hbm, buf, si0, si1, so0, so1):
    wid = lax.axis_index("s") * _NC + lax.axis_index("c")
    base = wid * _ROWS_PER_W
    in0 = pltpu.make_async_copy(u_hbm.at[4 * base], buf.at[0], si0)
    in1 = pltpu.make_async_copy(u_hbm.at[4 * (base + 1)], buf.at[1], si1)
    in0.start()
    in1.start()
    in0.wait()
    out0 = pltpu.make_async_copy(buf.at[0], out_hbm.at[base], so0)
    out0.start()
    in1.wait()
    out1 = pltpu.make_async_copy(buf.at[1], out_hbm.at[base + 1], so1)
    out1.start()
    out0.wait()
    out1.wait()


def kernel(u):
    return _sc_gather(u)


# speedup vs baseline: 10.9999x; 1.0138x over previous
import functools
import jax
import jax.numpy as jnp
from jax import lax
from jax.experimental import pallas as pl
from jax.experimental.pallas import tpu as pltpu
from jax.experimental.pallas import tpu_sc as plsc

_NC = 2
_NS = 16
_NW = _NC * _NS
_ROWS_OUT = 64
_ROWS_PER_W = _ROWS_OUT // _NW
_D = 32768


@functools.partial(
    pl.kernel,
    mesh=plsc.VectorSubcoreMesh(core_axis_name="c", subcore_axis_name="s"),
    out_type=jax.ShapeDtypeStruct((_ROWS_OUT, _D), jnp.float32),
    scratch_types=[
        pltpu.VMEM((4, _D // 2), jnp.float32),
        pltpu.SemaphoreType.DMA,
        pltpu.SemaphoreType.DMA,
        pltpu.SemaphoreType.DMA,
        pltpu.SemaphoreType.DMA,
        pltpu.SemaphoreType.DMA,
        pltpu.SemaphoreType.DMA,
        pltpu.SemaphoreType.DMA,
        pltpu.SemaphoreType.DMA,
    ],
)
def _sc_gather(u_hbm, out_hbm, buf, *sems):
    wid = lax.axis_index("s") * _NC + lax.axis_index("c")
    base = wid * _ROWS_PER_W
    half = _D // 2
    chunks = [(base + r, h) for r in range(_ROWS_PER_W) for h in range(2)]
    ins = []
    for k, (row, h) in enumerate(chunks):
        cp = pltpu.make_async_copy(
            u_hbm.at[4 * row, pl.ds(h * half, half)], buf.at[k], sems[k])
        cp.start()
        ins.append(cp)
    outs = []
    for k, (row, h) in enumerate(chunks):
        ins[k].wait()
        cp = pltpu.make_async_copy(
            buf.at[k], out_hbm.at[row, pl.ds(h * half, half)], sems[4 + k])
        cp.start()
        outs.append(cp)
    for cp in outs:
        cp.wait()


def kernel(u):
    return _sc_gather(u)


# speedup vs baseline: 11.0959x; 1.0087x over previous
import functools
import jax
import jax.numpy as jnp
from jax import lax
from jax.experimental import pallas as pl
from jax.experimental.pallas import tpu as pltpu
from jax.experimental.pallas import tpu_sc as plsc

_NC = 2
_NS = 16
_NW = _NC * _NS
_ROWS_OUT = 64
_ROWS_PER_W = _ROWS_OUT // _NW
_D = 32768


@functools.partial(
    pl.kernel,
    mesh=plsc.VectorSubcoreMesh(core_axis_name="c", subcore_axis_name="s"),
    out_type=jax.ShapeDtypeStruct((_ROWS_OUT, _D), jnp.float32),
    scratch_types=[
        pltpu.VMEM((8, _D // 4), jnp.float32),
        pltpu.SemaphoreType.DMA,
        pltpu.SemaphoreType.DMA,
        pltpu.SemaphoreType.DMA,
        pltpu.SemaphoreType.DMA,
        pltpu.SemaphoreType.DMA,
        pltpu.SemaphoreType.DMA,
        pltpu.SemaphoreType.DMA,
        pltpu.SemaphoreType.DMA,
        pltpu.SemaphoreType.DMA,
    ],
)
def _sc_gather(u_hbm, out_hbm, buf, *sems):
    wid = lax.axis_index("s") * _NC + lax.axis_index("c")
    base = wid * _ROWS_PER_W
    q = _D // 4
    chunks = [(base + r, h) for r in range(_ROWS_PER_W) for h in range(4)]
    ins = []
    for k, (row, h) in enumerate(chunks):
        cp = pltpu.make_async_copy(
            u_hbm.at[4 * row, pl.ds(h * q, q)], buf.at[k], sems[k])
        cp.start()
        ins.append(cp)
    outs = []
    for k, (row, h) in enumerate(chunks):
        ins[k].wait()
        cp = pltpu.make_async_copy(
            buf.at[k], out_hbm.at[row, pl.ds(h * q, q)], sems[8])
        cp.start()
        outs.append(cp)
    for cp in outs:
        cp.wait()


def kernel(u):
    return _sc_gather(u)
